# baseline (speedup 1.0000x reference)
import jax
import jax.numpy as jnp
from jax.experimental import pallas as pl

_OFFSETS = jnp.array(
    [[0, 0, 0], [1, 0, 0], [0, 1, 0], [0, 0, 1],
     [1, 1, 0], [0, 1, 1], [1, 0, 1], [1, 1, 1]],
    dtype=jnp.int32,
)


def _body(off_ref, inds_ref, feats_ref, oinds_ref, ofeats_ref):
    f = feats_ref[...]
    b, d = f.shape
    ofeats_ref[...] = jnp.broadcast_to(f[:, None, :], (b, 8, d))
    v = inds_ref[...]
    oinds_ref[...] = v[:, None, :] * 2 + off_ref[...][None]


def kernel(voxel_inds, feats):
    n, d = feats.shape
    B = 2000
    assert n % B == 0
    oinds, ofeats = pl.pallas_call(
        _body,
        grid=(n // B,),
        in_specs=[
            pl.BlockSpec((8, 3), lambda i: (0, 0)),
            pl.BlockSpec((B, 3), lambda i: (i, 0)),
            pl.BlockSpec((B, d), lambda i: (i, 0)),
        ],
        out_specs=[
            pl.BlockSpec((B, 8, 3), lambda i: (i, 0, 0)),
            pl.BlockSpec((B, 8, d), lambda i: (i, 0, 0)),
        ],
        out_shape=[
            jax.ShapeDtypeStruct((n, 8, 3), jnp.int32),
            jax.ShapeDtypeStruct((n, 8, d), jnp.float32),
        ],
    )(_OFFSETS, voxel_inds, feats)
    return oinds.reshape(-1, 3), ofeats.reshape(-1, d)
